# baseline (speedup 1.0000x reference)
import functools

import jax
import jax.numpy as jnp
from jax import lax
from jax.experimental import pallas as pl
from jax.experimental.pallas import tpu as pltpu
from jax.experimental.pallas import tpu_sc as plsc

N = 10000
E = 320000
G = 64
NP = 10240
NB = NP // 16
NC = 2
NW = 32
CH = 128
ER = E // CH
RPT = ER // NW
XTR = ER - RPT * NW

_MESH = dict(core_axis_name="c", subcore_axis_name="s")


def _sc_agg_build(D):
    K = 8 if D == 64 else 12
    NG = RPT // K
    R = RPT - NG * K

    @functools.partial(
        pl.kernel,
        mesh=plsc.VectorSubcoreMesh(**_MESH),
        compiler_params=pltpu.CompilerParams(use_tc_tiling_on_sc=False),
        out_type=jax.ShapeDtypeStruct((NC, NP, D), jnp.float32),
        scratch_types=[
            pltpu.VMEM(((RPT + 1) * CH,), jnp.int32),
            pltpu.VMEM((RPT + 1, CH), jnp.int32),
            [pltpu.VMEM((CH, D), jnp.float32) for _ in range(K)],
            pltpu.VMEM_SHARED((NP, D), jnp.float32),
            [pltpu.SemaphoreType.DMA for _ in range(K)],
            [pltpu.SemaphoreType.DMA for _ in range(K)],
        ],
    )
    def k(g_hbm, src_hbm, dst2_hbm, z_hbm, out_hbm,
          srcs, dsts, rows, acc, sg, ssem):
        c = lax.axis_index("c")
        s = lax.axis_index("s")
        wid = s * NC + c

        row0 = wid * RPT + jnp.minimum(wid, XTR)
        has_extra = wid < XTR
        pltpu.sync_copy(src_hbm.at[pl.ds(row0 * CH, RPT * CH)],
                        srcs.at[pl.ds(0, RPT * CH)])
        pltpu.sync_copy(dst2_hbm.at[pl.ds(row0, RPT)], dsts.at[pl.ds(0, RPT)])

        @pl.when(has_extra)
        def _():
            pltpu.sync_copy(src_hbm.at[pl.ds((row0 + RPT) * CH, CH)],
                            srcs.at[pl.ds(RPT * CH, CH)])
            pltpu.sync_copy(dst2_hbm.at[pl.ds(row0 + RPT, 1)],
                            dsts.at[pl.ds(RPT, 1)])

        pltpu.sync_copy(z_hbm, rows[0])
        for z in range(5):
            pltpu.sync_copy(rows[0], acc.at[pl.ds(s * NB + z * (NB // 5),
                                                  NB // 5)])
        plsc.subcore_barrier()

        def group_body(c0, n):
            hs = [pltpu.async_copy(
                      g_hbm.at[srcs.at[pl.ds((c0 + i) * CH, CH)]],
                      rows[i], sg[i]) for i in range(n)]
            ws = []
            for i in range(n):
                hs[i].wait()
                ws.append(pltpu.async_copy(rows[i], acc.at[dsts.at[c0 + i]],
                                           ssem[i], add=True))
            for w in ws:
                w.wait()

        def group(jg, carry):
            group_body(jg * K, K)
            return carry

        lax.fori_loop(0, NG, group, 0)
        if R:
            group_body(NG * K, R)

        @pl.when(has_extra)
        def _():
            pltpu.async_copy(g_hbm.at[srcs.at[pl.ds(RPT * CH, CH)]],
                             rows[0], sg[0]).wait()
            pltpu.sync_copy(rows[0], acc.at[dsts.at[RPT]], add=True)

        plsc.subcore_barrier()
        pltpu.sync_copy(acc.at[pl.ds(s * NB, NB)],
                        out_hbm.at[c, pl.ds(s * NB, NB)])

    return k


_sc_agg64 = _sc_agg_build(64)
_sc_agg32 = _sc_agg_build(32)


@functools.partial(
    pl.kernel,
    mesh=plsc.VectorSubcoreMesh(**_MESH),
    compiler_params=pltpu.CompilerParams(use_tc_tiling_on_sc=False),
    out_type=jax.ShapeDtypeStruct((NC, NP), jnp.float32),
    scratch_types=[
        pltpu.VMEM((RPT + 1, CH), jnp.int32),
        pltpu.VMEM((CH,), jnp.float32),
        pltpu.VMEM((NB,), jnp.float32),
        pltpu.VMEM_SHARED((NP,), jnp.float32),
    ],
)
def _sc_deg(dst2_hbm, ones_hbm, z_hbm, out_hbm, dsts, ones_v, zbuf, acc):
    c = lax.axis_index("c")
    s = lax.axis_index("s")
    wid = s * NC + c
    row0 = wid * RPT + jnp.minimum(wid, XTR)
    has_extra = wid < XTR

    pltpu.sync_copy(dst2_hbm.at[pl.ds(row0, RPT)], dsts.at[pl.ds(0, RPT)])

    @pl.when(has_extra)
    def _():
        pltpu.sync_copy(dst2_hbm.at[pl.ds(row0 + RPT, 1)],
                        dsts.at[pl.ds(RPT, 1)])

    pltpu.sync_copy(z_hbm, zbuf)
    pltpu.sync_copy(zbuf, acc.at[pl.ds(s * NB, NB)])
    pltpu.sync_copy(ones_hbm, ones_v)
    plsc.subcore_barrier()

    def step(j, carry):
        pltpu.sync_copy(ones_v, acc.at[dsts.at[j]], add=True)
        return carry

    lax.fori_loop(0, RPT, step, 0)

    @pl.when(has_extra)
    def _():
        pltpu.sync_copy(ones_v, acc.at[dsts.at[RPT]], add=True)

    plsc.subcore_barrier()
    pltpu.sync_copy(acc.at[pl.ds(s * NB, NB)], out_hbm.at[c, pl.ds(s * NB, NB)])


_BR = 2000


def _tc_mm1_body(x_ref, w_ref, hw_ref):
    hw_ref[...] = jnp.dot(x_ref[...], w_ref[...],
                          preferred_element_type=jnp.float32)


def _tc_mm1(x, W1):
    return pl.pallas_call(
        _tc_mm1_body,
        grid=(N // _BR,),
        in_specs=[
            pl.BlockSpec((_BR, 128), lambda i: (i, 0)),
            pl.BlockSpec((128, 64), lambda i: (0, 0)),
        ],
        out_specs=pl.BlockSpec((_BR, 64), lambda i: (i, 0)),
        out_shape=jax.ShapeDtypeStruct((N, 64), jnp.float32),
    )(x, W1)


def _tc_prep_body(hw_ref, d_ref, g_ref, dinv_ref):
    deg = d_ref[0] + d_ref[1] + 1.0
    dinv = lax.rsqrt(deg)
    g_ref[...] = hw_ref[...] * dinv
    dinv_ref[...] = dinv


def _tc_prep(hw1, deg_r):
    return pl.pallas_call(
        _tc_prep_body,
        grid=(N // _BR,),
        in_specs=[
            pl.BlockSpec((_BR, 64), lambda i: (i, 0)),
            pl.BlockSpec((2, _BR, 1), lambda i: (0, i, 0)),
        ],
        out_specs=(
            pl.BlockSpec((_BR, 64), lambda i: (i, 0)),
            pl.BlockSpec((_BR, 1), lambda i: (i, 0)),
        ),
        out_shape=(
            jax.ShapeDtypeStruct((N, 64), jnp.float32),
            jax.ShapeDtypeStruct((N, 1), jnp.float32),
        ),
    )(hw1, deg_r)


def _tc_layer_body(sp_ref, g_ref, dinv_ref, w_ref, b_ref, out_ref):
    dinv = dinv_ref[...]
    h = jnp.maximum(dinv * (sp_ref[0] + sp_ref[1] + g_ref[...]) + b_ref[...],
                    0.0)
    out_ref[...] = jnp.dot(h, w_ref[...],
                           preferred_element_type=jnp.float32) * dinv


def _tc_layer(S, g, dinv, W, b_r, dout):
    return pl.pallas_call(
        _tc_layer_body,
        grid=(N // _BR,),
        in_specs=[
            pl.BlockSpec((2, _BR, 64), lambda i: (0, i, 0)),
            pl.BlockSpec((_BR, 64), lambda i: (i, 0)),
            pl.BlockSpec((_BR, 1), lambda i: (i, 0)),
            pl.BlockSpec((64, dout), lambda i: (0, 0)),
            pl.BlockSpec((1, 64), lambda i: (0, 0)),
        ],
        out_specs=pl.BlockSpec((_BR, dout), lambda i: (i, 0)),
        out_shape=jax.ShapeDtypeStruct((N, dout), jnp.float32),
    )(S, g, dinv, W, b_r)


def _tc_head_body(sp_ref, g_ref, dinv_ref, b3_ref, batch_ref,
                  wc1, bc1, wc2, bc2, wc3, bc3, out_ref):
    h3 = dinv_ref[...] * (sp_ref[0, :N] + sp_ref[1, :N] + g_ref[...]) \
        + b3_ref[...]
    gids = lax.broadcasted_iota(jnp.int32, (G, N), 0)
    mask = (gids == batch_ref[...]).astype(jnp.float32)
    cnt = jnp.sum(mask, axis=1, keepdims=True)
    summ = jnp.dot(mask, h3, preferred_element_type=jnp.float32)
    emb = summ / jnp.maximum(cnt, 1.0)
    h = jnp.maximum(jnp.dot(emb, wc1[...],
                            preferred_element_type=jnp.float32) + bc1[...], 0.0)
    h = jnp.maximum(jnp.dot(h, wc2[...],
                            preferred_element_type=jnp.float32) + bc2[...], 0.0)
    out_ref[...] = jnp.dot(h, wc3[...],
                           preferred_element_type=jnp.float32) + bc3[...]


def _tc_head(S3, g3, dinv, b3_r, batch_r, Wc1, bc1_r, Wc2, bc2_r, Wc3, bc3_r):
    return pl.pallas_call(
        _tc_head_body,
        grid=(1,),
        in_specs=[
            pl.BlockSpec((2, NP, 32), lambda i: (0, 0, 0)),
            pl.BlockSpec((N, 32), lambda i: (0, 0)),
            pl.BlockSpec((N, 1), lambda i: (0, 0)),
            pl.BlockSpec((1, 32), lambda i: (0, 0)),
            pl.BlockSpec((1, N), lambda i: (0, 0)),
            pl.BlockSpec((32, 64), lambda i: (0, 0)),
            pl.BlockSpec((1, 64), lambda i: (0, 0)),
            pl.BlockSpec((64, 32), lambda i: (0, 0)),
            pl.BlockSpec((1, 32), lambda i: (0, 0)),
            pl.BlockSpec((32, 2), lambda i: (0, 0)),
            pl.BlockSpec((1, 2), lambda i: (0, 0)),
        ],
        out_specs=pl.BlockSpec((G, 2), lambda i: (0, 0)),
        out_shape=jax.ShapeDtypeStruct((G, 2), jnp.float32),
    )(S3, g3, dinv, b3_r, batch_r, Wc1, bc1_r, Wc2, bc2_r, Wc3, bc3_r)


def kernel(x, edge_index, batch, W1, b1, W2, b2, W3, b3,
           Wc1, bc1, Wc2, bc2, Wc3, bc3):
    f32 = jnp.float32
    z64 = jnp.zeros((NB // 5, 64), f32)
    z32 = jnp.zeros((NB // 5, 32), f32)
    z1 = jnp.zeros((NB,), f32)
    ones = jnp.ones((CH,), f32)
    src = edge_index[0]
    dst2 = edge_index[1].reshape(ER, CH)

    hw1 = _tc_mm1(x, W1)
    deg_p = _sc_deg(dst2, ones, z1)
    deg_r = deg_p.reshape(NC, NP, 1)
    g1, dinv = _tc_prep(hw1, deg_r)
    S1 = _sc_agg64(g1, src, dst2, z64)
    g2 = _tc_layer(S1, g1, dinv, W2, b1.reshape(1, 64), 64)
    S2 = _sc_agg64(g2, src, dst2, z64)
    g3 = _tc_layer(S2, g2, dinv, W3, b2.reshape(1, 64), 32)
    S3 = _sc_agg32(g3, src, dst2, z32)
    return _tc_head(S3, g3, dinv, b3.reshape(1, 32), batch.reshape(1, N),
                    Wc1, bc1.reshape(1, 64), Wc2, bc2.reshape(1, 32),
                    Wc3, bc3.reshape(1, 2))


# speedup vs baseline: 1.0268x; 1.0268x over previous
import functools

import jax
import jax.numpy as jnp
from jax import lax
from jax.experimental import pallas as pl
from jax.experimental.pallas import tpu as pltpu
from jax.experimental.pallas import tpu_sc as plsc

N = 10000
E = 320000
G = 64
NP = 10240
NB = NP // 16
NC = 2
NW = 32
CH = 128
ER = E // CH
RPT = ER // NW
XTR = ER - RPT * NW

_MESH = dict(core_axis_name="c", subcore_axis_name="s")


def _sc_agg_build(D):
    K = 8 if D == 64 else 12
    NG = RPT // K
    R = RPT - NG * K

    @functools.partial(
        pl.kernel,
        mesh=plsc.VectorSubcoreMesh(**_MESH),
        compiler_params=pltpu.CompilerParams(use_tc_tiling_on_sc=False),
        out_type=jax.ShapeDtypeStruct((NC, NP, D), jnp.float32),
        scratch_types=[
            pltpu.VMEM(((RPT + 1) * CH,), jnp.int32),
            pltpu.VMEM((RPT + 1, CH), jnp.int32),
            [pltpu.VMEM((CH, D), jnp.float32) for _ in range(K)],
            pltpu.VMEM_SHARED((NP, D), jnp.float32),
            [pltpu.SemaphoreType.DMA for _ in range(K)],
            [pltpu.SemaphoreType.DMA for _ in range(K)],
        ],
    )
    def k(g_hbm, src_hbm, dst2_hbm, z_hbm, out_hbm,
          srcs, dsts, rows, acc, sg, ssem):
        c = lax.axis_index("c")
        s = lax.axis_index("s")
        wid = s * NC + c

        row0 = wid * RPT + jnp.minimum(wid, XTR)
        has_extra = wid < XTR
        hsrc = pltpu.async_copy(src_hbm.at[pl.ds(row0 * CH, RPT * CH)],
                                srcs.at[pl.ds(0, RPT * CH)], sg[0])
        hdst = pltpu.async_copy(dst2_hbm.at[pl.ds(row0, RPT)],
                                dsts.at[pl.ds(0, RPT)], sg[1])

        @pl.when(has_extra)
        def _():
            pltpu.sync_copy(src_hbm.at[pl.ds((row0 + RPT) * CH, CH)],
                            srcs.at[pl.ds(RPT * CH, CH)])
            pltpu.sync_copy(dst2_hbm.at[pl.ds(row0 + RPT, 1)],
                            dsts.at[pl.ds(RPT, 1)])

        pltpu.sync_copy(z_hbm, rows[0])
        for z in range(5):
            pltpu.sync_copy(rows[0], acc.at[pl.ds(s * NB + z * (NB // 5),
                                                  NB // 5)])
        hsrc.wait()
        hdst.wait()
        plsc.subcore_barrier()

        def group_body(c0, n):
            hs = [pltpu.async_copy(
                      g_hbm.at[srcs.at[pl.ds((c0 + i) * CH, CH)]],
                      rows[i], sg[i]) for i in range(n)]
            ws = []
            for i in range(n):
                hs[i].wait()
                ws.append(pltpu.async_copy(rows[i], acc.at[dsts.at[c0 + i]],
                                           ssem[i], add=True))
            for w in ws:
                w.wait()

        def group(jg, carry):
            group_body(jg * K, K)
            return carry

        lax.fori_loop(0, NG, group, 0)
        if R:
            group_body(NG * K, R)

        @pl.when(has_extra)
        def _():
            pltpu.async_copy(g_hbm.at[srcs.at[pl.ds(RPT * CH, CH)]],
                             rows[0], sg[0]).wait()
            pltpu.sync_copy(rows[0], acc.at[dsts.at[RPT]], add=True)

        plsc.subcore_barrier()
        pltpu.sync_copy(acc.at[pl.ds(s * NB, NB)],
                        out_hbm.at[c, pl.ds(s * NB, NB)])

    return k


_sc_agg64 = _sc_agg_build(64)
_sc_agg32 = _sc_agg_build(32)


@functools.partial(
    pl.kernel,
    mesh=plsc.VectorSubcoreMesh(**_MESH),
    compiler_params=pltpu.CompilerParams(use_tc_tiling_on_sc=False),
    out_type=jax.ShapeDtypeStruct((NC, NP), jnp.float32),
    scratch_types=[
        pltpu.VMEM((RPT + 1, CH), jnp.int32),
        pltpu.VMEM((CH,), jnp.float32),
        pltpu.VMEM((NB,), jnp.float32),
        pltpu.VMEM_SHARED((NP,), jnp.float32),
        [pltpu.SemaphoreType.DMA for _ in range(6)],
    ],
)
def _sc_deg(dst2_hbm, ones_hbm, z_hbm, out_hbm, dsts, ones_v, zbuf, acc, sems):
    c = lax.axis_index("c")
    s = lax.axis_index("s")
    wid = s * NC + c
    row0 = wid * RPT + jnp.minimum(wid, XTR)
    has_extra = wid < XTR

    pltpu.sync_copy(dst2_hbm.at[pl.ds(row0, RPT)], dsts.at[pl.ds(0, RPT)])

    @pl.when(has_extra)
    def _():
        pltpu.sync_copy(dst2_hbm.at[pl.ds(row0 + RPT, 1)],
                        dsts.at[pl.ds(RPT, 1)])

    pltpu.sync_copy(z_hbm, zbuf)
    pltpu.sync_copy(zbuf, acc.at[pl.ds(s * NB, NB)])
    pltpu.sync_copy(ones_hbm, ones_v)
    plsc.subcore_barrier()

    def add_group(c0, n):
        hs = [pltpu.async_copy(ones_v, acc.at[dsts.at[c0 + i]], sems[i],
                               add=True) for i in range(n)]
        for h in hs:
            h.wait()

    def step(j, carry):
        add_group(j * 6, 6)
        return carry

    lax.fori_loop(0, RPT // 6, step, 0)

    @pl.when(has_extra)
    def _():
        pltpu.sync_copy(ones_v, acc.at[dsts.at[RPT]], add=True)

    plsc.subcore_barrier()
    pltpu.sync_copy(acc.at[pl.ds(s * NB, NB)], out_hbm.at[c, pl.ds(s * NB, NB)])


_BR = 2000


def _tc_prep_body(x_ref, w_ref, d_ref, g_ref, dinv_ref):
    deg = d_ref[0] + d_ref[1] + 1.0
    dinv = lax.rsqrt(deg)
    g_ref[...] = jnp.dot(x_ref[...], w_ref[...],
                         preferred_element_type=jnp.float32) * dinv
    dinv_ref[...] = dinv


def _tc_prep(x, W1, deg_r):
    return pl.pallas_call(
        _tc_prep_body,
        grid=(N // _BR,),
        in_specs=[
            pl.BlockSpec((_BR, 128), lambda i: (i, 0)),
            pl.BlockSpec((128, 64), lambda i: (0, 0)),
            pl.BlockSpec((2, _BR, 1), lambda i: (0, i, 0)),
        ],
        out_specs=(
            pl.BlockSpec((_BR, 64), lambda i: (i, 0)),
            pl.BlockSpec((_BR, 1), lambda i: (i, 0)),
        ),
        out_shape=(
            jax.ShapeDtypeStruct((N, 64), jnp.float32),
            jax.ShapeDtypeStruct((N, 1), jnp.float32),
        ),
    )(x, W1, deg_r)


def _tc_layer_body(sp_ref, g_ref, dinv_ref, w_ref, b_ref, out_ref):
    dinv = dinv_ref[...]
    h = jnp.maximum(dinv * (sp_ref[0] + sp_ref[1] + g_ref[...]) + b_ref[...],
                    0.0)
    out_ref[...] = jnp.dot(h, w_ref[...],
                           preferred_element_type=jnp.float32) * dinv


def _tc_layer(S, g, dinv, W, b_r, dout):
    return pl.pallas_call(
        _tc_layer_body,
        grid=(N // _BR,),
        in_specs=[
            pl.BlockSpec((2, _BR, 64), lambda i: (0, i, 0)),
            pl.BlockSpec((_BR, 64), lambda i: (i, 0)),
            pl.BlockSpec((_BR, 1), lambda i: (i, 0)),
            pl.BlockSpec((64, dout), lambda i: (0, 0)),
            pl.BlockSpec((1, 64), lambda i: (0, 0)),
        ],
        out_specs=pl.BlockSpec((_BR, dout), lambda i: (i, 0)),
        out_shape=jax.ShapeDtypeStruct((N, dout), jnp.float32),
    )(S, g, dinv, W, b_r)


def _tc_head_body(sp_ref, g_ref, dinv_ref, b3_ref, batch_ref,
                  wc1, bc1, wc2, bc2, wc3, bc3, out_ref):
    h3 = dinv_ref[...] * (sp_ref[0, :N] + sp_ref[1, :N] + g_ref[...]) \
        + b3_ref[...]
    gids = lax.broadcasted_iota(jnp.int32, (G, N), 0)
    mask = (gids == batch_ref[...]).astype(jnp.float32)
    cnt = jnp.sum(mask, axis=1, keepdims=True)
    summ = jnp.dot(mask, h3, preferred_element_type=jnp.float32)
    emb = summ / jnp.maximum(cnt, 1.0)
    h = jnp.maximum(jnp.dot(emb, wc1[...],
                            preferred_element_type=jnp.float32) + bc1[...], 0.0)
    h = jnp.maximum(jnp.dot(h, wc2[...],
                            preferred_element_type=jnp.float32) + bc2[...], 0.0)
    out_ref[...] = jnp.dot(h, wc3[...],
                           preferred_element_type=jnp.float32) + bc3[...]


def _tc_head(S3, g3, dinv, b3_r, batch_r, Wc1, bc1_r, Wc2, bc2_r, Wc3, bc3_r):
    return pl.pallas_call(
        _tc_head_body,
        grid=(1,),
        in_specs=[
            pl.BlockSpec((2, NP, 32), lambda i: (0, 0, 0)),
            pl.BlockSpec((N, 32), lambda i: (0, 0)),
            pl.BlockSpec((N, 1), lambda i: (0, 0)),
            pl.BlockSpec((1, 32), lambda i: (0, 0)),
            pl.BlockSpec((1, N), lambda i: (0, 0)),
            pl.BlockSpec((32, 64), lambda i: (0, 0)),
            pl.BlockSpec((1, 64), lambda i: (0, 0)),
            pl.BlockSpec((64, 32), lambda i: (0, 0)),
            pl.BlockSpec((1, 32), lambda i: (0, 0)),
            pl.BlockSpec((32, 2), lambda i: (0, 0)),
            pl.BlockSpec((1, 2), lambda i: (0, 0)),
        ],
        out_specs=pl.BlockSpec((G, 2), lambda i: (0, 0)),
        out_shape=jax.ShapeDtypeStruct((G, 2), jnp.float32),
    )(S3, g3, dinv, b3_r, batch_r, Wc1, bc1_r, Wc2, bc2_r, Wc3, bc3_r)


def kernel(x, edge_index, batch, W1, b1, W2, b2, W3, b3,
           Wc1, bc1, Wc2, bc2, Wc3, bc3):
    f32 = jnp.float32
    z64 = jnp.zeros((NB // 5, 64), f32)
    z32 = jnp.zeros((NB // 5, 32), f32)
    z1 = jnp.zeros((NB,), f32)
    ones = jnp.ones((CH,), f32)
    src = edge_index[0]
    dst2 = edge_index[1].reshape(ER, CH)

    deg_p = _sc_deg(dst2, ones, z1)
    deg_r = deg_p.reshape(NC, NP, 1)
    g1, dinv = _tc_prep(x, W1, deg_r)
    S1 = _sc_agg64(g1, src, dst2, z64)
    g2 = _tc_layer(S1, g1, dinv, W2, b1.reshape(1, 64), 64)
    S2 = _sc_agg64(g2, src, dst2, z64)
    g3 = _tc_layer(S2, g2, dinv, W3, b2.reshape(1, 64), 32)
    S3 = _sc_agg32(g3, src, dst2, z32)
    return _tc_head(S3, g3, dinv, b3.reshape(1, 32), batch.reshape(1, N),
                    Wc1, bc1.reshape(1, 64), Wc2, bc2.reshape(1, 32),
                    Wc3, bc3.reshape(1, 2))
